# Initial kernel scaffold; baseline (speedup 1.0000x reference)
import jax
import jax.numpy as jnp
from jax.experimental import pallas as pl


def kernel(x, h, d, T, b):
    raise NotImplementedError("write your pallas kernel here")


# speedup vs baseline: 7323.8282x; 7323.8282x over previous
import functools

import jax
import jax.numpy as jnp
from jax import lax
from jax.experimental import pallas as pl
from jax.experimental.pallas import tpu as pltpu
from jax.experimental.pallas import tpu_sc as plsc

NC = 2
NS = 16
NW = NC * NS
L = 16
K = 1024
CH = 32768


def _col_threshold(tbl_v, tc, c):
    tcb = jnp.full((L,), tc)
    ninf = jnp.full((L,), -jnp.inf, jnp.float32)
    pinf = jnp.full((L,), jnp.inf, jnp.float32)

    def body(j, carry):
        lmax, rmin = carry
        h1 = tbl_v[0, pl.ds(j * L, L)]
        hc = tbl_v[c, pl.ds(j * L, L)]
        below = hc < tcb
        lmax = jnp.maximum(lmax, jnp.where(below, h1, ninf))
        rmin = jnp.minimum(rmin, jnp.where(below, pinf, h1))
        return lmax, rmin

    lmax, rmin = lax.fori_loop(0, K // L, body, (ninf, pinf))
    return 0.5 * (jnp.max(lmax) + jnp.min(rmin))


def _sc_body(n, x_hbm, tbl_hbm, tv_hbm, dv_hbm, out_hbm,
             xbuf, obuf, tbl_v, tv_v, dv_v):
    wid = lax.axis_index("s") * NC + lax.axis_index("c")
    per = n // NW
    base = wid * per

    pltpu.sync_copy(tbl_hbm, tbl_v)
    pltpu.sync_copy(tv_hbm, tv_v)
    pltpu.sync_copy(dv_hbm, dv_v)

    tvec = tv_v[...]
    dvec = dv_v[...]
    b = tvec[4]
    t0 = _col_threshold(tbl_v, tvec[0], 0)
    t1 = tvec[1]
    t2 = _col_threshold(tbl_v, tvec[2], 2)
    t3 = _col_threshold(tbl_v, tvec[3], 3)

    t0v = jnp.full((L,), t0)
    t1v = jnp.full((L,), t1)
    t2v = jnp.full((L,), t2)
    t3v = jnp.full((L,), t3)
    d0bv = jnp.full((L,), dvec[0] - b)
    mbv = jnp.full((L,), -b)
    d1v = jnp.full((L,), dvec[1])
    d2v = jnp.full((L,), dvec[2])
    d3v = jnp.full((L,), dvec[3])
    zv = jnp.zeros((L,), jnp.float32)

    nch = per // CH
    for ch in range(nch):
        off = base + ch * CH
        pltpu.sync_copy(x_hbm.at[pl.ds(off, CH)], xbuf)

        @pl.loop(0, CH // L)
        def _compute(i):
            xv = xbuf[pl.ds(i * L, L)]
            acc = jnp.where(xv >= t0v, d0bv, mbv)
            acc = acc + jnp.where(xv >= t1v, d1v, zv)
            acc = acc + jnp.where(xv >= t2v, d2v, zv)
            acc = acc + jnp.where(xv >= t3v, d3v, zv)
            obuf[pl.ds(i * L, L)] = acc

        pltpu.sync_copy(obuf, out_hbm.at[pl.ds(off, CH)])


def kernel(x, h, d, T, b):
    n = x.shape[0]
    assert n % (NW * CH) == 0

    tbl = h.T.astype(jnp.float32)
    tv = jnp.zeros((L,), jnp.float32).at[:4].set(T).at[4].set(b)
    dv = jnp.zeros((L,), jnp.float32).at[:4].set(d)

    mesh = plsc.VectorSubcoreMesh(
        core_axis_name="c", subcore_axis_name="s",
        num_cores=NC, num_subcores=NS)
    run = pl.kernel(
        functools.partial(_sc_body, n),
        out_type=jax.ShapeDtypeStruct((n,), jnp.float32),
        mesh=mesh,
        compiler_params=pltpu.CompilerParams(needs_layout_passes=False),
        scratch_types=[
            pltpu.VMEM((CH,), jnp.float32),
            pltpu.VMEM((CH,), jnp.float32),
            pltpu.VMEM((4, K), jnp.float32),
            pltpu.VMEM((L,), jnp.float32),
            pltpu.VMEM((L,), jnp.float32),
        ],
    )
    return run(x, tbl, tv, dv)


# speedup vs baseline: 14295.4845x; 1.9519x over previous
import functools

import jax
import jax.numpy as jnp
from jax import lax
from jax.experimental import pallas as pl
from jax.experimental.pallas import tpu as pltpu
from jax.experimental.pallas import tpu_sc as plsc

NC = 2
NS = 16
NW = NC * NS
L = 16
K = 1024
CH = 16384
UNROLL = 8


def _col_threshold(tbl_v, tc, c):
    tcb = jnp.full((L,), tc)
    ninf = jnp.full((L,), -jnp.inf, jnp.float32)
    pinf = jnp.full((L,), jnp.inf, jnp.float32)

    def body(j, carry):
        lmax, rmin = carry
        h1 = tbl_v[0, pl.ds(j * L, L)]
        hc = tbl_v[c, pl.ds(j * L, L)]
        below = hc < tcb
        lmax = jnp.maximum(lmax, jnp.where(below, h1, ninf))
        rmin = jnp.minimum(rmin, jnp.where(below, pinf, h1))
        return lmax, rmin

    lmax, rmin = lax.fori_loop(0, K // L, body, (ninf, pinf))
    return 0.5 * (jnp.max(lmax) + jnp.min(rmin))


def _sc_body(n, x_hbm, tbl_hbm, tv_hbm, dv_hbm, out_hbm,
             xbuf0, xbuf1, obuf0, obuf1, tbl_v, tv_v, dv_v,
             isem0, isem1, osem0, osem1):
    wid = lax.axis_index("s") * NC + lax.axis_index("c")
    per = n // NW
    base = wid * per

    pltpu.sync_copy(tbl_hbm, tbl_v)
    pltpu.sync_copy(tv_hbm, tv_v)
    pltpu.sync_copy(dv_hbm, dv_v)

    tvec = tv_v[...]
    dvec = dv_v[...]
    b = tvec[4]
    t0 = _col_threshold(tbl_v, tvec[0], 0)
    t1 = tvec[1]
    t2 = _col_threshold(tbl_v, tvec[2], 2)
    t3 = _col_threshold(tbl_v, tvec[3], 3)

    t0v = jnp.full((L,), t0)
    t1v = jnp.full((L,), t1)
    t2v = jnp.full((L,), t2)
    t3v = jnp.full((L,), t3)
    d0bv = jnp.full((L,), dvec[0] - b)
    mbv = jnp.full((L,), -b)
    d1v = jnp.full((L,), dvec[1])
    d2v = jnp.full((L,), dvec[2])
    d3v = jnp.full((L,), dvec[3])
    zv = jnp.zeros((L,), jnp.float32)

    xbufs = (xbuf0, xbuf1)
    obufs = (obuf0, obuf1)
    isems = (isem0, isem1)
    osems = (osem0, osem1)
    nch = per // CH
    in_d = [None] * nch
    out_d = [None] * nch

    def start_in(ch):
        s = ch % 2
        in_d[ch] = pltpu.async_copy(
            x_hbm.at[pl.ds(base + ch * CH, CH)], xbufs[s], isems[s])

    def compute(xbuf, obuf):
        @plsc.parallel_loop(0, CH, step=L, unroll=UNROLL)
        def _compute(i):
            xv = xbuf[pl.ds(i, L)]
            acc = jnp.where(xv >= t0v, d0bv, mbv)
            acc = acc + jnp.where(xv >= t1v, d1v, zv)
            acc = acc + jnp.where(xv >= t2v, d2v, zv)
            acc = acc + jnp.where(xv >= t3v, d3v, zv)
            obuf[pl.ds(i, L)] = acc

    start_in(0)
    for ch in range(nch):
        s = ch % 2
        if ch + 1 < nch:
            start_in(ch + 1)
        in_d[ch].wait()
        if ch >= 2:
            out_d[ch - 2].wait()
        compute(xbufs[s], obufs[s])
        out_d[ch] = pltpu.async_copy(
            obufs[s], out_hbm.at[pl.ds(base + ch * CH, CH)], osems[s])
    out_d[nch - 2].wait()
    out_d[nch - 1].wait()


def kernel(x, h, d, T, b):
    n = x.shape[0]
    assert n % (NW * CH) == 0

    tbl = h.T.astype(jnp.float32)
    tv = jnp.zeros((L,), jnp.float32).at[:4].set(T).at[4].set(b)
    dv = jnp.zeros((L,), jnp.float32).at[:4].set(d)

    mesh = plsc.VectorSubcoreMesh(
        core_axis_name="c", subcore_axis_name="s",
        num_cores=NC, num_subcores=NS)
    run = pl.kernel(
        functools.partial(_sc_body, n),
        out_type=jax.ShapeDtypeStruct((n,), jnp.float32),
        mesh=mesh,
        compiler_params=pltpu.CompilerParams(needs_layout_passes=False),
        scratch_types=[
            pltpu.VMEM((CH,), jnp.float32),
            pltpu.VMEM((CH,), jnp.float32),
            pltpu.VMEM((CH,), jnp.float32),
            pltpu.VMEM((CH,), jnp.float32),
            pltpu.VMEM((4, K), jnp.float32),
            pltpu.VMEM((L,), jnp.float32),
            pltpu.VMEM((L,), jnp.float32),
            pltpu.SemaphoreType.DMA,
            pltpu.SemaphoreType.DMA,
            pltpu.SemaphoreType.DMA,
            pltpu.SemaphoreType.DMA,
        ],
    )
    return run(x, tbl, tv, dv)


# speedup vs baseline: 16566.3249x; 1.1589x over previous
import functools

import jax
import jax.numpy as jnp
from jax import lax
from jax.experimental import pallas as pl
from jax.experimental.pallas import tpu as pltpu
from jax.experimental.pallas import tpu_sc as plsc

NC = 2
NS = 16
NW = NC * NS
L = 16
K = 1024
CH = 16384
UNROLL = 8


def _col_threshold(tbl_v, tc, c):
    tcb = jnp.full((L,), tc)
    ninf = jnp.full((L,), -jnp.inf, jnp.float32)
    pinf = jnp.full((L,), jnp.inf, jnp.float32)

    def body(j, carry):
        lmax, rmin = carry
        h1 = tbl_v[0, pl.ds(j * L, L)]
        hc = tbl_v[c, pl.ds(j * L, L)]
        below = hc < tcb
        lmax = jnp.maximum(lmax, jnp.where(below, h1, ninf))
        rmin = jnp.minimum(rmin, jnp.where(below, pinf, h1))
        return lmax, rmin

    lmax, rmin = lax.fori_loop(0, K // L, body, (ninf, pinf))
    return 0.5 * (jnp.max(lmax) + jnp.min(rmin))


def _sc_body(n, x_hbm, tbl_hbm, tv_hbm, dv_hbm, out_hbm,
             xbuf0, xbuf1, obuf0, obuf1, tbl_v, tv_v, dv_v,
             isem0, isem1, osem0, osem1):
    wid = lax.axis_index("s") * NC + lax.axis_index("c")
    per = n // NW
    base = wid * per

    pltpu.sync_copy(tbl_hbm, tbl_v)
    pltpu.sync_copy(tv_hbm, tv_v)
    pltpu.sync_copy(dv_hbm, dv_v)

    tvec = tv_v[...]
    dvec = dv_v[...]
    b = tvec[4]
    pairs = [
        (_col_threshold(tbl_v, tvec[0], 0), dvec[0]),
        (tvec[1], dvec[1]),
        (_col_threshold(tbl_v, tvec[2], 2), dvec[2]),
        (_col_threshold(tbl_v, tvec[3], 3), dvec[3]),
    ]

    def cswap(i, j):
        ti, di = pairs[i]
        tj, dj = pairs[j]
        m = ti <= tj
        pairs[i] = (jnp.where(m, ti, tj), jnp.where(m, di, dj))
        pairs[j] = (jnp.where(m, tj, ti), jnp.where(m, dj, di))

    for i, j in ((0, 1), (2, 3), (0, 2), (1, 3), (1, 2)):
        cswap(i, j)

    s = -b
    sv = [jnp.full((L,), s)]
    for _, dc in pairs:
        s = s + dc
        sv.append(jnp.full((L,), s))
    tv = [jnp.full((L,), tc) for tc, _ in pairs]

    xbufs = (xbuf0, xbuf1)
    obufs = (obuf0, obuf1)
    isems = (isem0, isem1)
    osems = (osem0, osem1)
    nch = per // CH
    in_d = [None] * nch
    out_d = [None] * nch

    def start_in(ch):
        s = ch % 2
        in_d[ch] = pltpu.async_copy(
            x_hbm.at[pl.ds(base + ch * CH, CH)], xbufs[s], isems[s])

    def compute(xbuf, obuf):
        @plsc.parallel_loop(0, CH, step=L, unroll=UNROLL)
        def _compute(i):
            xv = xbuf[pl.ds(i, L)]
            hi = jnp.where(xv >= tv[3], sv[4], sv[3])
            hi = jnp.where(xv >= tv[2], hi, sv[2])
            lo = jnp.where(xv >= tv[0], sv[1], sv[0])
            obuf[pl.ds(i, L)] = jnp.where(xv >= tv[1], hi, lo)

    start_in(0)
    for ch in range(nch):
        s = ch % 2
        if ch + 1 < nch:
            start_in(ch + 1)
        in_d[ch].wait()
        if ch >= 2:
            out_d[ch - 2].wait()
        compute(xbufs[s], obufs[s])
        out_d[ch] = pltpu.async_copy(
            obufs[s], out_hbm.at[pl.ds(base + ch * CH, CH)], osems[s])
    out_d[nch - 2].wait()
    out_d[nch - 1].wait()


def kernel(x, h, d, T, b):
    n = x.shape[0]
    assert n % (NW * CH) == 0

    tbl = h.T.astype(jnp.float32)
    tv = jnp.zeros((L,), jnp.float32).at[:4].set(T).at[4].set(b)
    dv = jnp.zeros((L,), jnp.float32).at[:4].set(d)

    mesh = plsc.VectorSubcoreMesh(
        core_axis_name="c", subcore_axis_name="s",
        num_cores=NC, num_subcores=NS)
    run = pl.kernel(
        functools.partial(_sc_body, n),
        out_type=jax.ShapeDtypeStruct((n,), jnp.float32),
        mesh=mesh,
        compiler_params=pltpu.CompilerParams(needs_layout_passes=False),
        scratch_types=[
            pltpu.VMEM((CH,), jnp.float32),
            pltpu.VMEM((CH,), jnp.float32),
            pltpu.VMEM((CH,), jnp.float32),
            pltpu.VMEM((CH,), jnp.float32),
            pltpu.VMEM((4, K), jnp.float32),
            pltpu.VMEM((L,), jnp.float32),
            pltpu.VMEM((L,), jnp.float32),
            pltpu.SemaphoreType.DMA,
            pltpu.SemaphoreType.DMA,
            pltpu.SemaphoreType.DMA,
            pltpu.SemaphoreType.DMA,
        ],
    )
    return run(x, tbl, tv, dv)
